# baseline (device time: 25962 ns/iter reference)
import jax
import jax.numpy as jnp
from jax import lax
from jax.experimental import pallas as pl
from jax.experimental.pallas import tpu as pltpu

M = 2048
HALF = 1024
Q = 256
D = 1024
DH = D // 2
CHS = (32, 32, 64, 128)
OFFS = (0, 32, 64, 128)
NC = len(CHS)
NOUT = 4 * NC


def kernel(partial, gamma):
    gamma2 = gamma.reshape(1, D)

    def body(
        p_ref, g_ref, o_ref,
        sxf, lf, sx, rx, sS, rB, rA, rD, ob,
        cp_sems, lf_sem, ob_s,
        sx_s, rx_s, yS_s, rB_s, zS_s, rA_s, fA_s, dy_s, fB_s, dz_s,
    ):
        my_x = lax.axis_index("x")
        my_y = lax.axis_index("y")
        my_z = lax.axis_index("z")
        xn = (1 - my_x, my_y, my_z)
        yn = (my_x, 1 - my_y, my_z)
        zn = (my_x, my_y, 1 - my_z)

        k_me = 2 * my_y + my_z
        k_y = 2 * (1 - my_y) + my_z
        k_z = 2 * my_y + (1 - my_z)
        k_d = 2 * (1 - my_y) + (1 - my_z)

        send_base = (1 - my_x) * HALF + k_me * Q
        my_base = my_x * HALF + k_me * Q

        cps = []
        for c in range(NC):
            cp = pltpu.make_async_copy(
                p_ref.at[0, pl.ds(send_base + OFFS[c], CHS[c]), :],
                sxf.at[pl.ds(OFFS[c], CHS[c])],
                cp_sems.at[c],
            )
            cp.start()
            cps.append(cp)
        cpl = pltpu.make_async_copy(
            p_ref.at[0, pl.ds(my_base, Q), :], lf, lf_sem
        )
        cpl.start()

        barrier = pltpu.get_barrier_semaphore()
        for nbr in (xn, yn, zn):
            pl.semaphore_signal(
                barrier, inc=1, device_id=nbr,
                device_id_type=pl.DeviceIdType.MESH,
            )
        pl.semaphore_wait(barrier, 3)

        obs = []

        def store_out(rows_base, off, sz, values):
            ob[pl.ds(rows_base + off, sz), :] = values
            cp = pltpu.make_async_copy(
                ob.at[pl.ds(rows_base + off, sz)],
                o_ref.at[pl.ds(rows_base + off, sz)],
                ob_s.at[len(obs)],
            )
            cp.start()
            obs.append(cp)

        x_rdmas = []
        for c in range(NC):
            cps[c].wait()
            ch = pl.ds(OFFS[c], CHS[c])
            sx[ch, :] = sxf[ch, :].astype(jnp.bfloat16)
            r = pltpu.make_async_remote_copy(
                src_ref=sx.at[ch], dst_ref=rx.at[ch],
                send_sem=sx_s.at[c], recv_sem=rx_s.at[c],
                device_id=xn, device_id_type=pl.DeviceIdType.MESH,
            )
            r.start()
            x_rdmas.append(r)

        cpl.wait()

        yS, zS = [], []
        for c in range(NC):
            x_rdmas[c].wait_recv()
            ch = pl.ds(OFFS[c], CHS[c])
            s = rx[ch, :].astype(jnp.float32) + lf[ch, :]
            ms = jnp.mean(s * s, axis=1, keepdims=True) + 1e-6
            n = s * lax.rsqrt(ms) * g_ref[:, :]
            sS[ch, :] = n.astype(jnp.bfloat16)
            ry = pltpu.make_async_remote_copy(
                src_ref=sS.at[ch], dst_ref=rB.at[ch],
                send_sem=yS_s.at[c], recv_sem=rB_s.at[c],
                device_id=yn, device_id_type=pl.DeviceIdType.MESH,
            )
            ry.start()
            yS.append(ry)
            rz_ = pltpu.make_async_remote_copy(
                src_ref=sS.at[ch], dst_ref=rA.at[ch],
                send_sem=zS_s.at[c], recv_sem=rA_s.at[c],
                device_id=zn, device_id_type=pl.DeviceIdType.MESH,
            )
            rz_.start()
            zS.append(rz_)
            store_out(k_me * Q, OFFS[c], CHS[c], n)

        fA, fB = [], []
        for c in range(NC):
            ch = pl.ds(OFFS[c], CHS[c])
            yS[c].wait_recv()
            r = pltpu.make_async_remote_copy(
                src_ref=rB.at[ch, pl.ds(DH, DH)],
                dst_ref=rD.at[ch, pl.ds(DH, DH)],
                send_sem=fB_s.at[c], recv_sem=dz_s.at[c],
                device_id=zn, device_id_type=pl.DeviceIdType.MESH,
            )
            r.start()
            fB.append(r)
            store_out(k_y * Q, OFFS[c], CHS[c], rB[ch, :].astype(jnp.float32))

            zS[c].wait_recv()
            r = pltpu.make_async_remote_copy(
                src_ref=rA.at[ch, pl.ds(0, DH)],
                dst_ref=rD.at[ch, pl.ds(0, DH)],
                send_sem=fA_s.at[c], recv_sem=dy_s.at[c],
                device_id=yn, device_id_type=pl.DeviceIdType.MESH,
            )
            r.start()
            fA.append(r)
            store_out(k_z * Q, OFFS[c], CHS[c], rA[ch, :].astype(jnp.float32))

        for c in range(NC):
            fA[c].wait_recv()
            fB[c].wait_recv()
            ch = pl.ds(OFFS[c], CHS[c])
            store_out(k_d * Q, OFFS[c], CHS[c], rD[ch, :].astype(jnp.float32))

        for c in range(NC):
            x_rdmas[c].wait_send()
            yS[c].wait_send()
            zS[c].wait_send()
            fA[c].wait_send()
            fB[c].wait_send()
        for cp in obs:
            cp.wait()

    return pl.pallas_call(
        body,
        out_shape=jax.ShapeDtypeStruct((HALF, D), jnp.float32),
        in_specs=[
            pl.BlockSpec(memory_space=pl.ANY),
            pl.BlockSpec(memory_space=pltpu.VMEM),
        ],
        out_specs=pl.BlockSpec(memory_space=pl.ANY),
        scratch_shapes=[
            pltpu.VMEM((Q, D), jnp.float32),
            pltpu.VMEM((Q, D), jnp.float32),
            pltpu.VMEM((Q, D), jnp.bfloat16),
            pltpu.VMEM((Q, D), jnp.bfloat16),
            pltpu.VMEM((Q, D), jnp.bfloat16),
            pltpu.VMEM((Q, D), jnp.bfloat16),
            pltpu.VMEM((Q, D), jnp.bfloat16),
            pltpu.VMEM((Q, D), jnp.bfloat16),
            pltpu.VMEM((HALF, D), jnp.float32),
            pltpu.SemaphoreType.DMA((NC,)),
            pltpu.SemaphoreType.DMA,
            pltpu.SemaphoreType.DMA((NOUT,)),
            pltpu.SemaphoreType.DMA((NC,)),
            pltpu.SemaphoreType.DMA((NC,)),
            pltpu.SemaphoreType.DMA((NC,)),
            pltpu.SemaphoreType.DMA((NC,)),
            pltpu.SemaphoreType.DMA((NC,)),
            pltpu.SemaphoreType.DMA((NC,)),
            pltpu.SemaphoreType.DMA((NC,)),
            pltpu.SemaphoreType.DMA((NC,)),
            pltpu.SemaphoreType.DMA((NC,)),
            pltpu.SemaphoreType.DMA((NC,)),
        ],
        compiler_params=pltpu.CompilerParams(collective_id=0),
    )(partial, gamma2)


# device time: 25715 ns/iter; 1.0096x vs baseline; 1.0096x over previous
import jax
import jax.numpy as jnp
from jax import lax
from jax.experimental import pallas as pl
from jax.experimental.pallas import tpu as pltpu

M = 2048
HALF = 1024
Q = 256
D = 1024
CHS = (32, 32, 64, 128)
OFFS = (0, 32, 64, 128)
NC = len(CHS)


def kernel(partial, gamma):
    gamma2 = gamma.reshape(1, D)

    def body(
        p_ref, g_ref, o_ref,
        sxf, lf, sx, rx, sS, rB, rA, rD,
        cp_sems, lf_sem,
        sx_s, rx_s, yS_s, rB_s, zS_s, rA_s, fA_s, dy_s, fB_s, dz_s,
    ):
        my_x = lax.axis_index("x")
        my_y = lax.axis_index("y")
        my_z = lax.axis_index("z")
        xn = (1 - my_x, my_y, my_z)
        yn = (my_x, 1 - my_y, my_z)
        zn = (my_x, my_y, 1 - my_z)

        k_me = 2 * my_y + my_z
        k_y = 2 * (1 - my_y) + my_z
        k_z = 2 * my_y + (1 - my_z)
        k_d = 2 * (1 - my_y) + (1 - my_z)

        send_base = (1 - my_x) * HALF + k_me * Q
        my_base = my_x * HALF + k_me * Q

        cps = []
        for c in range(NC):
            cp = pltpu.make_async_copy(
                p_ref.at[0, pl.ds(send_base + OFFS[c], CHS[c]), :],
                sxf.at[pl.ds(OFFS[c], CHS[c])],
                cp_sems.at[c],
            )
            cp.start()
            cps.append(cp)
        cpl = pltpu.make_async_copy(
            p_ref.at[0, pl.ds(my_base, Q), :], lf, lf_sem
        )
        cpl.start()

        barrier = pltpu.get_barrier_semaphore()
        for nbr in (xn, yn, zn):
            pl.semaphore_signal(
                barrier, inc=1, device_id=nbr,
                device_id_type=pl.DeviceIdType.MESH,
            )
        pl.semaphore_wait(barrier, 3)

        x_rdmas = []
        for c in range(NC):
            cps[c].wait()
            ch = pl.ds(OFFS[c], CHS[c])
            sx[ch, :] = sxf[ch, :].astype(jnp.bfloat16)
            r = pltpu.make_async_remote_copy(
                src_ref=sx.at[ch], dst_ref=rx.at[ch],
                send_sem=sx_s.at[c], recv_sem=rx_s.at[c],
                device_id=xn, device_id_type=pl.DeviceIdType.MESH,
            )
            r.start()
            x_rdmas.append(r)

        cpl.wait()

        yS, zS = [], []
        for c in range(NC):
            x_rdmas[c].wait_recv()
            ch = pl.ds(OFFS[c], CHS[c])
            s = rx[ch, :].astype(jnp.float32) + lf[ch, :]
            ms = jnp.mean(s * s, axis=1, keepdims=True) + 1e-6
            n = s * lax.rsqrt(ms) * g_ref[:, :]
            sS[ch, :] = n.astype(jnp.bfloat16)
            ry = pltpu.make_async_remote_copy(
                src_ref=sS.at[ch], dst_ref=rB.at[ch],
                send_sem=yS_s.at[c], recv_sem=rB_s.at[c],
                device_id=yn, device_id_type=pl.DeviceIdType.MESH,
            )
            ry.start()
            yS.append(ry)
            rz_ = pltpu.make_async_remote_copy(
                src_ref=sS.at[ch], dst_ref=rA.at[ch],
                send_sem=zS_s.at[c], recv_sem=rA_s.at[c],
                device_id=zn, device_id_type=pl.DeviceIdType.MESH,
            )
            rz_.start()
            zS.append(rz_)
            o_ref[pl.ds(k_me * Q + OFFS[c], CHS[c]), :] = n

        DH = D // 2
        fA, fB = [], []
        for c in range(NC):
            ch = pl.ds(OFFS[c], CHS[c])
            yS[c].wait_recv()
            r = pltpu.make_async_remote_copy(
                src_ref=rB.at[ch, pl.ds(DH, DH)],
                dst_ref=rD.at[ch, pl.ds(DH, DH)],
                send_sem=fB_s.at[c], recv_sem=dz_s.at[c],
                device_id=zn, device_id_type=pl.DeviceIdType.MESH,
            )
            r.start()
            fB.append(r)
            o_ref[pl.ds(k_y * Q + OFFS[c], CHS[c]), :] = rB[ch, :].astype(
                jnp.float32
            )

            zS[c].wait_recv()
            r = pltpu.make_async_remote_copy(
                src_ref=rA.at[ch, pl.ds(0, DH)],
                dst_ref=rD.at[ch, pl.ds(0, DH)],
                send_sem=fA_s.at[c], recv_sem=dy_s.at[c],
                device_id=yn, device_id_type=pl.DeviceIdType.MESH,
            )
            r.start()
            fA.append(r)
            o_ref[pl.ds(k_z * Q + OFFS[c], CHS[c]), :] = rA[ch, :].astype(
                jnp.float32
            )

        for c in range(NC):
            fA[c].wait_recv()
            fB[c].wait_recv()
            ch = pl.ds(OFFS[c], CHS[c])
            o_ref[pl.ds(k_d * Q + OFFS[c], CHS[c]), :] = rD[ch, :].astype(
                jnp.float32
            )

        for c in range(NC):
            x_rdmas[c].wait_send()
            yS[c].wait_send()
            zS[c].wait_send()
            fA[c].wait_send()
            fB[c].wait_send()

    return pl.pallas_call(
        body,
        out_shape=jax.ShapeDtypeStruct((HALF, D), jnp.float32),
        in_specs=[
            pl.BlockSpec(memory_space=pl.ANY),
            pl.BlockSpec(memory_space=pltpu.VMEM),
        ],
        out_specs=pl.BlockSpec(memory_space=pltpu.VMEM),
        scratch_shapes=[
            pltpu.VMEM((Q, D), jnp.float32),
            pltpu.VMEM((Q, D), jnp.float32),
            pltpu.VMEM((Q, D), jnp.bfloat16),
            pltpu.VMEM((Q, D), jnp.bfloat16),
            pltpu.VMEM((Q, D), jnp.bfloat16),
            pltpu.VMEM((Q, D), jnp.bfloat16),
            pltpu.VMEM((Q, D), jnp.bfloat16),
            pltpu.VMEM((Q, D), jnp.bfloat16),
            pltpu.SemaphoreType.DMA((NC,)),
            pltpu.SemaphoreType.DMA,
            pltpu.SemaphoreType.DMA((NC,)),
            pltpu.SemaphoreType.DMA((NC,)),
            pltpu.SemaphoreType.DMA((NC,)),
            pltpu.SemaphoreType.DMA((NC,)),
            pltpu.SemaphoreType.DMA((NC,)),
            pltpu.SemaphoreType.DMA((NC,)),
            pltpu.SemaphoreType.DMA((NC,)),
            pltpu.SemaphoreType.DMA((NC,)),
            pltpu.SemaphoreType.DMA((NC,)),
            pltpu.SemaphoreType.DMA((NC,)),
        ],
        compiler_params=pltpu.CompilerParams(collective_id=0),
    )(partial, gamma2)


# device time: 23057 ns/iter; 1.1260x vs baseline; 1.1153x over previous
import jax
import jax.numpy as jnp
from jax import lax
from jax.experimental import pallas as pl
from jax.experimental.pallas import tpu as pltpu

M = 2048
HALF = 1024
Q = 256
D = 1024
NC = 4
QCH = Q // NC
NH = NC // 2


def kernel(partial, gamma):
    gamma2 = gamma.reshape(1, D)

    def body(
        p_ref, g_ref, o_ref,
        sxf, lf, sx, rx, sS, rB, rA, rD,
        cp_sems, lf_sem,
        sx_s, rx_s, yS_s, rB_s, zS_s, rA_s, fA_s, dy_s, fB_s, dz_s,
        yzbar,
    ):
        my_x = lax.axis_index("x")
        my_y = lax.axis_index("y")
        my_z = lax.axis_index("z")
        xn = (1 - my_x, my_y, my_z)
        yn = (my_x, 1 - my_y, my_z)
        zn = (my_x, my_y, 1 - my_z)

        k_me = 2 * my_y + my_z
        k_y = 2 * (1 - my_y) + my_z
        k_z = 2 * my_y + (1 - my_z)
        k_d = 2 * (1 - my_y) + (1 - my_z)

        send_base = (1 - my_x) * HALF + k_me * Q
        my_base = my_x * HALF + k_me * Q

        cps = []
        for c in range(NC):
            cp = pltpu.make_async_copy(
                p_ref.at[0, pl.ds(send_base + c * QCH, QCH), :],
                sxf.at[pl.ds(c * QCH, QCH)],
                cp_sems.at[c],
            )
            cp.start()
            cps.append(cp)
        cpl = pltpu.make_async_copy(
            p_ref.at[0, pl.ds(my_base, Q), :], lf, lf_sem
        )
        cpl.start()

        barrier = pltpu.get_barrier_semaphore()
        pl.semaphore_signal(
            barrier, inc=1, device_id=xn,
            device_id_type=pl.DeviceIdType.MESH,
        )
        for nbr in (yn, zn):
            pl.semaphore_signal(
                yzbar, inc=1, device_id=nbr,
                device_id_type=pl.DeviceIdType.MESH,
            )
        pl.semaphore_wait(barrier, 1)

        x_rdmas = []
        for c in range(NC):
            cps[c].wait()
            ch = pl.ds(c * QCH, QCH)
            sx[ch, :] = sxf[ch, :].astype(jnp.bfloat16)
            r = pltpu.make_async_remote_copy(
                src_ref=sx.at[ch], dst_ref=rx.at[ch],
                send_sem=sx_s.at[c], recv_sem=rx_s.at[c],
                device_id=xn, device_id_type=pl.DeviceIdType.MESH,
            )
            r.start()
            x_rdmas.append(r)

        cpl.wait()
        pl.semaphore_wait(yzbar, 2)

        yS, zS = [], []
        for c in range(NC):
            x_rdmas[c].wait_recv()
            ch = pl.ds(c * QCH, QCH)
            s = rx[ch, :].astype(jnp.float32) + lf[ch, :]
            ms = jnp.mean(s * s, axis=1, keepdims=True) + 1e-6
            n = s * lax.rsqrt(ms) * g_ref[:, :]
            sS[ch, :] = n.astype(jnp.bfloat16)
            ry = pltpu.make_async_remote_copy(
                src_ref=sS.at[ch], dst_ref=rB.at[ch],
                send_sem=yS_s.at[c], recv_sem=rB_s.at[c],
                device_id=yn, device_id_type=pl.DeviceIdType.MESH,
            )
            ry.start()
            yS.append(ry)
            rz_ = pltpu.make_async_remote_copy(
                src_ref=sS.at[ch], dst_ref=rA.at[ch],
                send_sem=zS_s.at[c], recv_sem=rA_s.at[c],
                device_id=zn, device_id_type=pl.DeviceIdType.MESH,
            )
            rz_.start()
            zS.append(rz_)
            o_ref[pl.ds(k_me * Q + c * QCH, QCH), :] = n

        DH = D // 2
        fA, fB = [], []
        for c in range(NC):
            ch = pl.ds(c * QCH, QCH)
            yS[c].wait_recv()
            r = pltpu.make_async_remote_copy(
                src_ref=rB.at[ch, pl.ds(DH, DH)],
                dst_ref=rD.at[ch, pl.ds(DH, DH)],
                send_sem=fB_s.at[c], recv_sem=dz_s.at[c],
                device_id=zn, device_id_type=pl.DeviceIdType.MESH,
            )
            r.start()
            fB.append(r)
            o_ref[pl.ds(k_y * Q + c * QCH, QCH), :] = rB[ch, :].astype(
                jnp.float32
            )

            zS[c].wait_recv()
            r = pltpu.make_async_remote_copy(
                src_ref=rA.at[ch, pl.ds(0, DH)],
                dst_ref=rD.at[ch, pl.ds(0, DH)],
                send_sem=fA_s.at[c], recv_sem=dy_s.at[c],
                device_id=yn, device_id_type=pl.DeviceIdType.MESH,
            )
            r.start()
            fA.append(r)
            o_ref[pl.ds(k_z * Q + c * QCH, QCH), :] = rA[ch, :].astype(
                jnp.float32
            )

        for c in range(NC):
            fA[c].wait_recv()
            fB[c].wait_recv()
            ch = pl.ds(c * QCH, QCH)
            o_ref[pl.ds(k_d * Q + c * QCH, QCH), :] = rD[ch, :].astype(
                jnp.float32
            )

        for c in range(NC):
            x_rdmas[c].wait_send()
            yS[c].wait_send()
            zS[c].wait_send()
            fA[c].wait_send()
            fB[c].wait_send()

    return pl.pallas_call(
        body,
        out_shape=jax.ShapeDtypeStruct((HALF, D), jnp.float32),
        in_specs=[
            pl.BlockSpec(memory_space=pl.ANY),
            pl.BlockSpec(memory_space=pltpu.VMEM),
        ],
        out_specs=pl.BlockSpec(memory_space=pltpu.VMEM),
        scratch_shapes=[
            pltpu.VMEM((Q, D), jnp.float32),
            pltpu.VMEM((Q, D), jnp.float32),
            pltpu.VMEM((Q, D), jnp.bfloat16),
            pltpu.VMEM((Q, D), jnp.bfloat16),
            pltpu.VMEM((Q, D), jnp.bfloat16),
            pltpu.VMEM((Q, D), jnp.bfloat16),
            pltpu.VMEM((Q, D), jnp.bfloat16),
            pltpu.VMEM((Q, D), jnp.bfloat16),
            pltpu.SemaphoreType.DMA((NC,)),
            pltpu.SemaphoreType.DMA,
            pltpu.SemaphoreType.DMA((NC,)),
            pltpu.SemaphoreType.DMA((NC,)),
            pltpu.SemaphoreType.DMA((NC,)),
            pltpu.SemaphoreType.DMA((NC,)),
            pltpu.SemaphoreType.DMA((NC,)),
            pltpu.SemaphoreType.DMA((NC,)),
            pltpu.SemaphoreType.DMA((NC,)),
            pltpu.SemaphoreType.DMA((NC,)),
            pltpu.SemaphoreType.DMA((NC,)),
            pltpu.SemaphoreType.DMA((NC,)),
            pltpu.SemaphoreType.REGULAR,
        ],
        compiler_params=pltpu.CompilerParams(collective_id=0),
    )(partial, gamma2)
